# baseline (speedup 1.0000x reference)
import jax
import jax.numpy as jnp
from jax.experimental import pallas as pl

_BATCH_BLOCK = 64


def _add_kernel(x_ref, pos_ref, out_ref):
    out_ref[...] = x_ref[...] + pos_ref[...]


def kernel(x, pos_embedding):
    batch, seq_len, embed_dim = x.shape
    flat = seq_len * embed_dim
    x2 = x.reshape(batch, flat)
    pos2 = pos_embedding.reshape(1, flat)
    bb = _BATCH_BLOCK
    grid = (batch // bb,)
    out = pl.pallas_call(
        _add_kernel,
        grid=grid,
        in_specs=[
            pl.BlockSpec((bb, flat), lambda i: (i, 0)),
            pl.BlockSpec((1, flat), lambda i: (0, 0)),
        ],
        out_specs=pl.BlockSpec((bb, flat), lambda i: (i, 0)),
        out_shape=jax.ShapeDtypeStruct((batch, flat), x.dtype),
    )(x2, pos2)
    return out.reshape(batch, seq_len, embed_dim)


# speedup vs baseline: 1.0009x; 1.0009x over previous
import jax
import jax.numpy as jnp
from jax.experimental import pallas as pl
from jax.experimental.pallas import tpu as pltpu

_BATCH_BLOCK = 64


def _add_kernel(x_ref, pos_ref, out_ref):
    out_ref[...] = x_ref[...] + pos_ref[...]


def kernel(x, pos_embedding):
    batch, seq_len, embed_dim = x.shape
    flat = seq_len * embed_dim
    x2 = x.reshape(batch, flat)
    pos2 = pos_embedding.reshape(1, flat)
    bb = _BATCH_BLOCK
    grid = (batch // bb,)
    out = pl.pallas_call(
        _add_kernel,
        grid=grid,
        in_specs=[
            pl.BlockSpec((bb, flat), lambda i: (i, 0)),
            pl.BlockSpec((1, flat), lambda i: (0, 0)),
        ],
        out_specs=pl.BlockSpec((bb, flat), lambda i: (i, 0)),
        out_shape=jax.ShapeDtypeStruct((batch, flat), x.dtype),
        compiler_params=pltpu.CompilerParams(
            dimension_semantics=("parallel",)),
    )(x2, pos2)
    return out.reshape(batch, seq_len, embed_dim)


# speedup vs baseline: 1.0075x; 1.0066x over previous
import jax
import jax.numpy as jnp
from jax.experimental import pallas as pl
from jax.experimental.pallas import tpu as pltpu

_BB = 64
_NBUF = 8
_DEPTH = 4


def _stream_kernel(x_ref, pos_ref, out_ref, bufs, in_sems, out_sems):
    batch = x_ref.shape[0]
    n_chunks = batch // _BB

    def in_copy(c):
        return pltpu.make_async_copy(
            x_ref.at[pl.ds(c * _BB, _BB), :], bufs.at[c % _NBUF],
            in_sems.at[c % _NBUF])

    def out_copy(c):
        return pltpu.make_async_copy(
            bufs.at[c % _NBUF], out_ref.at[pl.ds(c * _BB, _BB), :],
            out_sems.at[c % _NBUF])

    for c in range(_DEPTH):
        in_copy(c).start()

    pos = pos_ref[...]
    for c in range(n_chunks):
        slot = c % _NBUF
        in_copy(c).wait()
        bufs[slot] = bufs[slot] + pos
        out_copy(c).start()
        nc = c + _DEPTH
        if nc < n_chunks:
            if nc >= _NBUF:
                out_copy(nc - _NBUF).wait()
            in_copy(nc).start()
    for c in range(max(0, n_chunks - _NBUF), n_chunks):
        out_copy(c).wait()


def kernel(x, pos_embedding):
    batch, seq_len, embed_dim = x.shape
    flat = seq_len * embed_dim
    x2 = x.reshape(batch, flat)
    pos2 = pos_embedding.reshape(1, flat)
    out = pl.pallas_call(
        _stream_kernel,
        in_specs=[
            pl.BlockSpec(memory_space=pltpu.HBM),
            pl.BlockSpec(memory_space=pltpu.VMEM),
        ],
        out_specs=pl.BlockSpec(memory_space=pltpu.HBM),
        out_shape=jax.ShapeDtypeStruct((batch, flat), x.dtype),
        scratch_shapes=[
            pltpu.VMEM((_NBUF, _BB, flat), jnp.float32),
            pltpu.SemaphoreType.DMA((_NBUF,)),
            pltpu.SemaphoreType.DMA((_NBUF,)),
        ],
    )(x2, pos2)
    return out.reshape(batch, seq_len, embed_dim)
